# baseline (speedup 1.0000x reference)
import jax
import jax.numpy as jnp
from jax.experimental import pallas as pl


def _body(x_ref, o_ref):
    x = x_ref[...]
    nxt = jnp.roll(x, -1, axis=1)
    prv = jnp.roll(x, 1, axis=1)
    lane = jax.lax.broadcasted_iota(jnp.int32, x.shape, 1)
    even = (lane & 1) == 0
    zero = jnp.zeros_like(x)
    even_out = jnp.where((x >= 0) | (nxt > -x), x, zero)
    odd_out = jnp.where(prv >= 0, x, jnp.maximum(x, zero))
    o_ref[...] = jnp.where(even, even_out, odd_out)


def kernel(bounds):
    n = bounds.shape[0]
    C = 2048
    R = (2 * n) // C
    BR = 256
    x2 = bounds.reshape(R, C)
    out = pl.pallas_call(
        _body,
        grid=(R // BR,),
        in_specs=[pl.BlockSpec((BR, C), lambda i: (i, 0))],
        out_specs=pl.BlockSpec((BR, C), lambda i: (i, 0)),
        out_shape=jax.ShapeDtypeStruct((R, C), bounds.dtype),
    )(x2)
    return out.reshape(n, 2)


# speedup vs baseline: 1.0338x; 1.0338x over previous
import jax
import jax.numpy as jnp
from jax.experimental import pallas as pl


def _body(x_ref, o_ref):
    x = x_ref[...]
    nxt = jnp.roll(x, -1, axis=1)
    prv = jnp.roll(x, 1, axis=1)
    lane = jax.lax.broadcasted_iota(jnp.int32, x.shape, 1)
    even = (lane & 1) == 0
    zero = jnp.zeros_like(x)
    even_out = jnp.where((x >= 0) | (nxt > -x), x, zero)
    odd_out = jnp.where(prv >= 0, x, jnp.maximum(x, zero))
    o_ref[...] = jnp.where(even, even_out, odd_out)


def kernel(bounds):
    n = bounds.shape[0]
    C = 128
    R = (2 * n) // C
    BR = 4096
    x2 = bounds.reshape(R, C)
    out = pl.pallas_call(
        _body,
        grid=(R // BR,),
        in_specs=[pl.BlockSpec((BR, C), lambda i: (i, 0))],
        out_specs=pl.BlockSpec((BR, C), lambda i: (i, 0)),
        out_shape=jax.ShapeDtypeStruct((R, C), bounds.dtype),
    )(x2)
    return out.reshape(n, 2)
